# baseline (speedup 1.0000x reference)
import functools

import jax
import jax.numpy as jnp
from jax import lax
from jax.experimental import pallas as pl
from jax.experimental.pallas import tpu as pltpu
from jax.experimental.pallas import tpu_sc as plsc

_D = 16
_DE = 4
_YW = 5 * _D
_NC = 2
_NS = 16
_NW = _NC * _NS
_CH = 512
_EP = 163840
_NCH = _EP // (_NW * _CH)
_NP = 10016


def _tc_pre(x_indivi, x_event, c1, c2):
    n_i, n_e = x_indivi.shape[0], x_event.shape[0]

    def body(xi, xe, c1r, c2r, y1o, y2o):
        y1o[0:n_e, :] = jnp.dot(xe[...], c1r[...], preferred_element_type=jnp.float32)
        y1o[n_e:_NP, :] = jnp.zeros((_NP - n_e, _YW), jnp.float32)
        y2o[0:n_i, :] = jnp.dot(xi[...], c2r[...], preferred_element_type=jnp.float32)
        y2o[n_i:_NP, :] = jnp.zeros((_NP - n_i, _YW), jnp.float32)

    return pl.pallas_call(
        body,
        out_shape=[
            jax.ShapeDtypeStruct((_NP, _YW), jnp.float32),
            jax.ShapeDtypeStruct((_NP, _YW), jnp.float32),
        ],
    )(x_indivi, x_event, c1, c2)


def _sc_layer1(y1, s1, d1, a1c, y2, s2, d2, a2c, n_i, n_e):
    rpt_i = n_i // _NS
    rpt_e = n_e // _NS
    mesh = plsc.VectorSubcoreMesh(core_axis_name="c", subcore_axis_name="s")

    @functools.partial(
        pl.kernel,
        out_type=[
            jax.ShapeDtypeStruct((_NW, rpt_i, _D), jnp.float32),
            jax.ShapeDtypeStruct((_NW, rpt_e, _D), jnp.float32),
        ],
        mesh=mesh,
        compiler_params=pltpu.CompilerParams(use_tc_tiling_on_sc=False,
                                             needs_layout_passes=False),
        scratch_types=[
            pltpu.VMEM((2, _CH), jnp.int32),
            pltpu.VMEM((2, _CH), jnp.int32),
            pltpu.VMEM((_DE, _CH), jnp.float32),
            pltpu.VMEM((2, _CH, _YW), jnp.float32),
            pltpu.VMEM((2, _CH, _D), jnp.float32),
            pltpu.VMEM_SHARED((n_i, _D), jnp.float32),
            pltpu.VMEM_SHARED((n_e, _D), jnp.float32),
            pltpu.SemaphoreType.DMA,
            pltpu.SemaphoreType.DMA,
            pltpu.SemaphoreType.DMA,
            pltpu.SemaphoreType.DMA,
            pltpu.SemaphoreType.DMA,
            pltpu.SemaphoreType.DMA,
            pltpu.SemaphoreType.DMA,
        ],
    )
    def k(y1h, s1h, d1h, a1h0, a1h1, a1h2, a1h3,
          y2h, s2h, d2h, a2h0, a2h1, a2h2, a2h3,
          aggi_h, agge_h,
          src_v, dst_v, a_v, rows_v, msg_v, aggi_sh, agge_sh,
          sl0, sl1, sg0, sg1, sda, sc0, sc1):
        c = lax.axis_index("c")
        s = lax.axis_index("s")
        wid = c * _NS + s
        sload = (sl0, sl1)
        sgat = (sg0, sg1)
        ssc = (sc0, sc1)

        def zero_body(i, carry):
            msg_v[0, i] = jnp.zeros((_D,), jnp.float32)
            return carry

        lax.fori_loop(0, _CH, zero_body, 0)
        for rpt, agg_sh in ((rpt_i, aggi_sh), (rpt_e, agge_sh)):
            for r0 in range(0, rpt, _CH):
                w = min(_CH, rpt - r0)
                pltpu.sync_copy(msg_v.at[0, pl.ds(0, w)],
                                agg_sh.at[pl.ds(s * rpt + r0, w)])
        plsc.subcore_barrier()

        def do_type(yh, sh, dh, ahs, agg_sh):
            def start_src(row, b):
                pltpu.async_copy(sh.at[row], src_v.at[b], sload[b])

            def wait_src(b):
                pltpu.make_async_copy(sh.at[0], src_v.at[b], sload[b]).wait()

            def start_gather(b):
                pltpu.async_copy(yh.at[src_v.at[b]], rows_v.at[b], sgat[b])

            def wait_gather(b):
                pltpu.make_async_copy(yh.at[pl.ds(0, _CH)], rows_v.at[b],
                                      sgat[b]).wait()

            base = wid * _NCH
            start_src(base, 0)
            wait_src(0)
            start_gather(0)
            start_src(base + 1, 1)

            def wait_scatter(b):
                pltpu.make_async_copy(msg_v.at[b],
                                      agg_sh.at[dst_v.at[b]], ssc[b]).wait()

            def pair(j, carry):
                for b in (0, 1):
                    cc = 2 * j + b
                    row = base + cc

                    @pl.when(cc >= 2)
                    def _():
                        wait_scatter(b)

                    pltpu.async_copy(dh.at[row], dst_v.at[b], sda)
                    for kk in range(_DE):
                        pltpu.async_copy(ahs[kk].at[row], a_v.at[kk], sda)

                    @pl.when(cc + 1 < _NCH)
                    def _():
                        wait_src(1 - b)
                        start_gather(1 - b)

                    wait_gather(b)

                    @pl.when(cc + 2 < _NCH)
                    def _():
                        start_src(row + 2, b)

                    pltpu.make_async_copy(dh.at[0], dst_v.at[b], sda).wait()
                    for kk in range(_DE):
                        pltpu.make_async_copy(ahs[kk].at[0], a_v.at[kk],
                                              sda).wait()

                    def grp(g, carry2):
                        av0 = a_v[0, pl.ds(g * 16, 16)]
                        av1 = a_v[1, pl.ds(g * 16, 16)]
                        av2 = a_v[2, pl.ds(g * 16, 16)]
                        av3 = a_v[3, pl.ds(g * 16, 16)]
                        for t in range(16):
                            i = g * 16 + t
                            msg_v[b, i] = (
                                av0[t] * rows_v[b, i, pl.ds(0, _D)]
                                + av1[t] * rows_v[b, i, pl.ds(_D, _D)]
                                + av2[t] * rows_v[b, i, pl.ds(2 * _D, _D)]
                                + av3[t] * rows_v[b, i, pl.ds(3 * _D, _D)]
                                + rows_v[b, i, pl.ds(4 * _D, _D)])
                        return carry2

                    lax.fori_loop(0, _CH // 16, grp, 0, unroll=4)
                    pltpu.async_copy(msg_v.at[b], agg_sh.at[dst_v.at[b]],
                                     ssc[b], add=True)
                return carry

            lax.fori_loop(0, _NCH // 2, pair, 0)
            wait_scatter(0)
            wait_scatter(1)

        do_type(y1h, s1h, d1h, (a1h0, a1h1, a1h2, a1h3), aggi_sh)
        do_type(y2h, s2h, d2h, (a2h0, a2h1, a2h2, a2h3), agge_sh)
        plsc.subcore_barrier()
        pltpu.sync_copy(aggi_sh.at[pl.ds(s * rpt_i, rpt_i)], aggi_h.at[wid])
        pltpu.sync_copy(agge_sh.at[pl.ds(s * rpt_e, rpt_e)], agge_h.at[wid])

    return k(y1, s1, d1, a1c[0], a1c[1], a1c[2], a1c[3],
             y2, s2, d2, a2c[0], a2c[1], a2c[2], a2c[3])


def _tc_mid(aggi_p, agge_p, xip, xep, bd_r1, bd_r2, b1row, b2row,
            bd_c3, bd_w3, b3row):
    npk = xip.shape[0]

    def body(ai, ae, xir, xer, w1r, w2r, b1r, b2r, c3r, w3r, b3r, y3o, r3o):
        hi = jax.nn.sigmoid(
            ai[0:npk, :] + ai[npk:2 * npk, :] + b1r[...]
            + jnp.dot(xir[...], w1r[...], preferred_element_type=jnp.float32))
        he = jax.nn.sigmoid(
            ae[0:npk, :] + ae[npk:2 * npk, :] + b2r[...]
            + jnp.dot(xer[...], w2r[...], preferred_element_type=jnp.float32))
        y3o[0:npk, :] = jnp.dot(he, c3r[...], preferred_element_type=jnp.float32)
        y3o[npk:npk + 2, :] = jnp.zeros((2, 8 * _D), jnp.float32)
        r3o[...] = jnp.dot(hi, w3r[...], preferred_element_type=jnp.float32) + b3r[...]

    return pl.pallas_call(
        body,
        out_shape=[
            jax.ShapeDtypeStruct((npk + 2, 8 * _D), jnp.float32),
            jax.ShapeDtypeStruct((npk, 8), jnp.float32),
        ],
    )(aggi_p, agge_p, xip, xep, bd_r1, bd_r2, b1row, b2row, bd_c3, bd_w3, b3row)


def _sc_layer2(y3p, s3, d3, a3c, n_i):
    rpt_i = n_i // _NS
    mesh = plsc.VectorSubcoreMesh(core_axis_name="c", subcore_axis_name="s")

    @functools.partial(
        pl.kernel,
        out_type=jax.ShapeDtypeStruct((_NW, rpt_i, _D), jnp.float32),
        mesh=mesh,
        compiler_params=pltpu.CompilerParams(use_tc_tiling_on_sc=False,
                                             needs_layout_passes=False),
        scratch_types=[
            pltpu.VMEM((2, _CH), jnp.int32),
            pltpu.VMEM((2, _CH), jnp.int32),
            pltpu.VMEM((_DE, _CH), jnp.float32),
            pltpu.VMEM((2, _CH, _D), jnp.float32),
            pltpu.VMEM((2, _CH, _D), jnp.float32),
            pltpu.VMEM_SHARED((n_i, _D), jnp.float32),
            pltpu.SemaphoreType.DMA,
            pltpu.SemaphoreType.DMA,
            pltpu.SemaphoreType.DMA,
            pltpu.SemaphoreType.DMA,
            pltpu.SemaphoreType.DMA,
            pltpu.SemaphoreType.DMA,
            pltpu.SemaphoreType.DMA,
        ],
    )
    def k(y3h, s3h, d3h, a3h0, a3h1, a3h2, a3h3, agg_h,
          src_v, dst_v, a_v, rows_v, msg_v, agg_sh,
          sl0, sl1, sg0, sg1, sda, sc0, sc1):
        c = lax.axis_index("c")
        s = lax.axis_index("s")
        wid = c * _NS + s
        sload = (sl0, sl1)
        sgat = (sg0, sg1)
        ssc = (sc0, sc1)

        def zero_body(i, carry):
            msg_v[0, i] = jnp.zeros((_D,), jnp.float32)
            msg_v[1, i] = jnp.zeros((_D,), jnp.float32)
            return carry

        lax.fori_loop(0, _CH, zero_body, 0)
        for r0 in range(0, rpt_i, _CH):
            w = min(_CH, rpt_i - r0)
            pltpu.sync_copy(msg_v.at[0, pl.ds(0, w)],
                            agg_sh.at[pl.ds(s * rpt_i + r0, w)])
        plsc.subcore_barrier()

        lanes = lax.iota(jnp.int32, _D)
        col0 = jnp.zeros((_D,), jnp.int32)

        def start_src(row, b):
            pltpu.async_copy(s3h.at[row], src_v.at[b], sload[b])

        def wait_src(b):
            pltpu.make_async_copy(s3h.at[0], src_v.at[b], sload[b]).wait()

        def start_gather(b):
            pltpu.async_copy(y3h.at[src_v.at[b]], rows_v.at[b], sgat[b])

        def wait_gather(b):
            pltpu.make_async_copy(y3h.at[pl.ds(0, _CH)], rows_v.at[b],
                                  sgat[b]).wait()

        ahs = (a3h0, a3h1, a3h2, a3h3)
        base = wid * _NCH
        start_src(base, 0)
        wait_src(0)
        start_gather(0)
        start_src(base + 1, 1)

        def wait_scatter(b):
            pltpu.make_async_copy(msg_v.at[b],
                                  agg_sh.at[dst_v.at[b]], ssc[b]).wait()

        def pair(j, carry):
            for b in (0, 1):
                cc = 2 * j + b
                row = base + cc

                @pl.when(cc >= 2)
                def _():
                    wait_scatter(b)

                pltpu.async_copy(d3h.at[row], dst_v.at[b], sda)
                for kk in range(_DE):
                    pltpu.async_copy(ahs[kk].at[row], a_v.at[kk], sda)

                @pl.when(cc + 1 < _NCH)
                def _():
                    wait_src(1 - b)
                    start_gather(1 - b)

                wait_gather(b)

                @pl.when(cc + 2 < _NCH)
                def _():
                    start_src(row + 2, b)

                pltpu.make_async_copy(d3h.at[0], dst_v.at[b], sda).wait()
                for kk in range(_DE):
                    pltpu.make_async_copy(ahs[kk].at[0], a_v.at[kk], sda).wait()

                def grp(g, carry2):
                    ev = g * _D + lanes
                    av0 = a_v[0, pl.ds(g * 16, 16)]
                    av1 = a_v[1, pl.ds(g * 16, 16)]
                    av2 = a_v[2, pl.ds(g * 16, 16)]
                    av3 = a_v[3, pl.ds(g * 16, 16)]
                    y0 = plsc.load_gather(rows_v, [col0 + b, ev, col0])
                    y1_ = plsc.load_gather(rows_v, [col0 + b, ev, col0 + 1])
                    y2_ = plsc.load_gather(rows_v, [col0 + b, ev, col0 + 2])
                    y3_ = plsc.load_gather(rows_v, [col0 + b, ev, col0 + 3])
                    y4_ = plsc.load_gather(rows_v, [col0 + b, ev, col0 + 4])
                    m = av0 * y0 + av1 * y1_ + av2 * y2_ + av3 * y3_ + y4_
                    plsc.store_scatter(msg_v, [col0 + b, ev, col0], m)
                    return carry2

                lax.fori_loop(0, _CH // _D, grp, 0)
                pltpu.async_copy(msg_v.at[b], agg_sh.at[dst_v.at[b]],
                                 ssc[b], add=True)
            return carry

        lax.fori_loop(0, _NCH // 2, pair, 0)
        wait_scatter(0)
        wait_scatter(1)

        plsc.subcore_barrier()
        pltpu.sync_copy(agg_sh.at[pl.ds(s * rpt_i, rpt_i)], agg_h.at[wid])

    return k(y3p, s3, d3, a3c[0], a3c[1], a3c[2], a3c[3])


def _tc_final(agg3_p, r3p, sel):
    npk = r3p.shape[0]

    def body(a3, r3r, selr, outo):
        m = jnp.dot(a3[0:npk, :] + a3[npk:2 * npk, :], selr[...],
                    preferred_element_type=jnp.float32)
        outo[...] = jax.nn.sigmoid(m + r3r[...])

    return pl.pallas_call(
        body,
        out_shape=jax.ShapeDtypeStruct((npk, 8), jnp.float32),
    )(agg3_p, r3p, sel)


_PB = 16384


def _tc_prep(ei1, ei2, a1t, a2t, n_i, n_e):
    e = ei1.shape[1]
    npad = _NP - 10000

    def body(ei1r, ei2r, a1r, a2r,
             s1o, d1o, s2o, d2o,
             a1o0, a1o1, a1o2, a1o3, a2o0, a2o1, a2o2, a2o3):
        i = pl.program_id(0)
        gcol = i * _PB + lax.broadcasted_iota(jnp.int32, (_PB,), 0)
        real = gcol < e
        padi = gcol - e
        s1o[...] = jnp.where(real, ei1r[0, :], n_e + padi % npad)
        d1o[...] = jnp.where(real, ei1r[1, :], padi % n_i)
        s2o[...] = jnp.where(real, ei2r[0, :], n_i + padi % npad)
        d2o[...] = jnp.where(real, ei2r[1, :], padi % n_e)
        for k, (o1, o2) in enumerate(((a1o0, a2o0), (a1o1, a2o1),
                                      (a1o2, a2o2), (a1o3, a2o3))):
            o1[...] = jnp.where(real, a1r[k, :], 0.0)
            o2[...] = jnp.where(real, a2r[k, :], 0.0)

    outs = pl.pallas_call(
        body,
        grid=(_EP // _PB,),
        in_specs=[pl.BlockSpec((2, _PB), lambda i: (0, i)),
                  pl.BlockSpec((2, _PB), lambda i: (0, i)),
                  pl.BlockSpec((_DE, _PB), lambda i: (0, i)),
                  pl.BlockSpec((_DE, _PB), lambda i: (0, i))],
        out_specs=[pl.BlockSpec((_PB,), lambda i: (i,))] * 12,
        out_shape=([jax.ShapeDtypeStruct((_EP,), jnp.int32)] * 4
                   + [jax.ShapeDtypeStruct((_EP,), jnp.float32)] * 8),
    )(ei1, ei2, a1t, a2t)
    blk = [o.reshape(_EP // _CH, _CH) for o in outs]
    return blk[0], blk[1], blk[2], blk[3], tuple(blk[4:8]), tuple(blk[8:12])


def kernel(x_indivi, x_event, edge_index_e2i, edge_attr_e2i, edge_index_i2e,
           edge_attr_i2e, nnW1, nnb1, rootW1, b1, nnW2, nnb2, rootW2, b2,
           nnW3, nnb3, rootW3, b3):
    n_i, n_e = x_indivi.shape[0], x_event.shape[0]

    m1 = nnW1.reshape(_D, _D, _DE)
    c1 = jnp.concatenate([m1[:, :, k] for k in range(_DE)]
                         + [nnb1.reshape(_D, _D)], axis=1)
    m2 = nnW2.reshape(_D, _D, _DE)
    c2 = jnp.concatenate([m2[:, :, k] for k in range(_DE)]
                         + [nnb2.reshape(_D, _D)], axis=1)
    c3 = jnp.concatenate([nnW3, nnb3.reshape(_D, 1),
                          jnp.zeros((_D, _D - _DE - 1), jnp.float32)], axis=1)

    s1, d1, s2, d2, a1c, a2c = _tc_prep(
        edge_index_e2i.astype(jnp.int32), edge_index_i2e.astype(jnp.int32),
        edge_attr_e2i.T, edge_attr_i2e.T, n_i, n_e)

    npk = n_i // 8
    eye8 = jnp.eye(8, dtype=jnp.float32)
    bd_r1 = jnp.kron(eye8, rootW1.T)
    bd_r2 = jnp.kron(eye8, rootW2.T)
    bd_c3 = jnp.kron(eye8, c3)
    bd_w3 = jnp.kron(eye8, rootW3.T)
    sel3 = jnp.kron(eye8, jnp.eye(_D, 1, dtype=jnp.float32))
    b1row = jnp.tile(b1, 8).reshape(1, 8 * _D)
    b2row = jnp.tile(b2, 8).reshape(1, 8 * _D)
    b3row = jnp.tile(b3, 8).reshape(1, 8)

    y1, y2 = _tc_pre(x_indivi, x_event, c1, c2)
    xip = x_indivi.reshape(npk, 8 * _D)
    xep = x_event.reshape(n_e // 8, 8 * _D)

    aggi, agge = _sc_layer1(y1, s1, d1, a1c, y2, s2, d2, a2c, n_i, n_e)
    aggi_p = aggi.reshape(_NC * npk, 8 * _D)
    agge_p = agge.reshape(_NC * (n_e // 8), 8 * _D)

    y3pk, r3p = _tc_mid(aggi_p, agge_p, xip, xep, bd_r1, bd_r2,
                        b1row, b2row, bd_c3, bd_w3, b3row)
    y3p = y3pk.reshape(_NP, _D)

    agg3_p = _sc_layer2(y3p, s1, d1, a1c, n_i).reshape(_NC * npk, 8 * _D)
    return _tc_final(agg3_p, r3p, sel3).reshape(n_i, 1)


# speedup vs baseline: 1.0735x; 1.0735x over previous
import functools

import jax
import jax.numpy as jnp
from jax import lax
from jax.experimental import pallas as pl
from jax.experimental.pallas import tpu as pltpu
from jax.experimental.pallas import tpu_sc as plsc

_D = 16
_DE = 4
_YW = 5 * _D
_NC = 2
_NS = 16
_NW = _NC * _NS
_CH = 512
_EP = 163840
_NCH = _EP // (_NW * _CH)
_NP = 10016


def _tc_y(x, cmat):
    n = x.shape[0]

    def body(xr, cr, yo):
        yo[0:n, :] = jnp.dot(xr[...], cr[...], preferred_element_type=jnp.float32)
        yo[n:_NP, :] = jnp.zeros((_NP - n, _YW), jnp.float32)

    return pl.pallas_call(
        body,
        out_shape=jax.ShapeDtypeStruct((_NP, _YW), jnp.float32),
    )(x, cmat)


def _sc_edge_pass(y, s_in, d_in, acols, n_dst):
    rpt = n_dst // _NS
    mesh = plsc.VectorSubcoreMesh(core_axis_name="c", subcore_axis_name="s")

    @functools.partial(
        pl.kernel,
        out_type=jax.ShapeDtypeStruct((_NW, rpt, _D), jnp.float32),
        mesh=mesh,
        compiler_params=pltpu.CompilerParams(use_tc_tiling_on_sc=False,
                                             needs_layout_passes=False),
        scratch_types=[
            pltpu.VMEM((2, _CH), jnp.int32),
            pltpu.VMEM((2, _CH), jnp.int32),
            pltpu.VMEM((_DE, _CH), jnp.float32),
            pltpu.VMEM((2, _CH, _YW), jnp.float32),
            pltpu.VMEM((2, _CH, _D), jnp.float32),
            pltpu.VMEM_SHARED((n_dst, _D), jnp.float32),
            pltpu.SemaphoreType.DMA,
            pltpu.SemaphoreType.DMA,
            pltpu.SemaphoreType.DMA,
            pltpu.SemaphoreType.DMA,
            pltpu.SemaphoreType.DMA,
            pltpu.SemaphoreType.DMA,
            pltpu.SemaphoreType.DMA,
        ],
    )
    def k(yh, sh, dh, ah0, ah1, ah2, ah3, agg_h,
          src_v, dst_v, a_v, rows_v, msg_v, agg_sh,
          sl0, sl1, sg0, sg1, sda, sc0, sc1):
        c = lax.axis_index("c")
        s = lax.axis_index("s")
        wid = c * _NS + s
        sload = (sl0, sl1)
        sgat = (sg0, sg1)
        ssc = (sc0, sc1)
        ahs = (ah0, ah1, ah2, ah3)

        def zero_body(i, carry):
            msg_v[0, i] = jnp.zeros((_D,), jnp.float32)
            return carry

        lax.fori_loop(0, _CH, zero_body, 0)
        for r0 in range(0, rpt, _CH):
            w = min(_CH, rpt - r0)
            pltpu.sync_copy(msg_v.at[0, pl.ds(0, w)],
                            agg_sh.at[pl.ds(s * rpt + r0, w)])
        plsc.subcore_barrier()

        if True:
            def start_src(row, b):
                pltpu.async_copy(sh.at[row], src_v.at[b], sload[b])

            def wait_src(b):
                pltpu.make_async_copy(sh.at[0], src_v.at[b], sload[b]).wait()

            def start_gather(b):
                pltpu.async_copy(yh.at[src_v.at[b]], rows_v.at[b], sgat[b])

            def wait_gather(b):
                pltpu.make_async_copy(yh.at[pl.ds(0, _CH)], rows_v.at[b],
                                      sgat[b]).wait()

            base = wid * _NCH
            start_src(base, 0)
            wait_src(0)
            start_gather(0)
            start_src(base + 1, 1)

            def wait_scatter(b):
                pltpu.make_async_copy(msg_v.at[b],
                                      agg_sh.at[dst_v.at[b]], ssc[b]).wait()

            def pair(j, carry):
                for b in (0, 1):
                    cc = 2 * j + b
                    row = base + cc

                    @pl.when(cc >= 2)
                    def _():
                        wait_scatter(b)

                    pltpu.async_copy(dh.at[row], dst_v.at[b], sda)
                    for kk in range(_DE):
                        pltpu.async_copy(ahs[kk].at[row], a_v.at[kk], sda)

                    @pl.when(cc + 1 < _NCH)
                    def _():
                        wait_src(1 - b)
                        start_gather(1 - b)

                    wait_gather(b)

                    @pl.when(cc + 2 < _NCH)
                    def _():
                        start_src(row + 2, b)

                    pltpu.make_async_copy(dh.at[0], dst_v.at[b], sda).wait()
                    for kk in range(_DE):
                        pltpu.make_async_copy(ahs[kk].at[0], a_v.at[kk],
                                              sda).wait()

                    def grp(g, carry2):
                        av0 = a_v[0, pl.ds(g * 16, 16)]
                        av1 = a_v[1, pl.ds(g * 16, 16)]
                        av2 = a_v[2, pl.ds(g * 16, 16)]
                        av3 = a_v[3, pl.ds(g * 16, 16)]
                        for t in range(16):
                            i = g * 16 + t
                            msg_v[b, i] = (
                                av0[t] * rows_v[b, i, pl.ds(0, _D)]
                                + av1[t] * rows_v[b, i, pl.ds(_D, _D)]
                                + av2[t] * rows_v[b, i, pl.ds(2 * _D, _D)]
                                + av3[t] * rows_v[b, i, pl.ds(3 * _D, _D)]
                                + rows_v[b, i, pl.ds(4 * _D, _D)])
                        return carry2

                    lax.fori_loop(0, _CH // 16, grp, 0)
                    pltpu.async_copy(msg_v.at[b], agg_sh.at[dst_v.at[b]],
                                     ssc[b], add=True)
                return carry

            lax.fori_loop(0, _NCH // 2, pair, 0)
            wait_scatter(0)
            wait_scatter(1)

        plsc.subcore_barrier()
        pltpu.sync_copy(agg_sh.at[pl.ds(s * rpt, rpt)], agg_h.at[wid])

    return k(y, s_in, d_in, acols[0], acols[1], acols[2], acols[3])


def _tc_mid(aggi_p, agge_p, xip, xep, bd_r1, bd_r2, b1row, b2row,
            bd_c3, bd_w3, b3row):
    npk = xip.shape[0]

    def body(ai, ae, xir, xer, w1r, w2r, b1r, b2r, c3r, w3r, b3r, y3o, r3o):
        hi = jax.nn.sigmoid(
            ai[0:npk, :] + ai[npk:2 * npk, :] + b1r[...]
            + jnp.dot(xir[...], w1r[...], preferred_element_type=jnp.float32))
        he = jax.nn.sigmoid(
            ae[0:npk, :] + ae[npk:2 * npk, :] + b2r[...]
            + jnp.dot(xer[...], w2r[...], preferred_element_type=jnp.float32))
        y3o[0:npk, :] = jnp.dot(he, c3r[...], preferred_element_type=jnp.float32)
        y3o[npk:npk + 2, :] = jnp.zeros((2, 8 * _D), jnp.float32)
        r3o[...] = jnp.dot(hi, w3r[...], preferred_element_type=jnp.float32) + b3r[...]

    return pl.pallas_call(
        body,
        out_shape=[
            jax.ShapeDtypeStruct((npk + 2, 8 * _D), jnp.float32),
            jax.ShapeDtypeStruct((npk, 8), jnp.float32),
        ],
    )(aggi_p, agge_p, xip, xep, bd_r1, bd_r2, b1row, b2row, bd_c3, bd_w3, b3row)


def _sc_layer2(y3p, s3, d3, a3c, n_i):
    rpt_i = n_i // _NS
    mesh = plsc.VectorSubcoreMesh(core_axis_name="c", subcore_axis_name="s")

    @functools.partial(
        pl.kernel,
        out_type=jax.ShapeDtypeStruct((_NW, rpt_i, _D), jnp.float32),
        mesh=mesh,
        compiler_params=pltpu.CompilerParams(use_tc_tiling_on_sc=False,
                                             needs_layout_passes=False),
        scratch_types=[
            pltpu.VMEM((2, _CH), jnp.int32),
            pltpu.VMEM((2, _CH), jnp.int32),
            pltpu.VMEM((_DE, _CH), jnp.float32),
            pltpu.VMEM((2, _CH, _D), jnp.float32),
            pltpu.VMEM((2, _CH, _D), jnp.float32),
            pltpu.VMEM_SHARED((n_i, _D), jnp.float32),
            pltpu.SemaphoreType.DMA,
            pltpu.SemaphoreType.DMA,
            pltpu.SemaphoreType.DMA,
            pltpu.SemaphoreType.DMA,
            pltpu.SemaphoreType.DMA,
            pltpu.SemaphoreType.DMA,
            pltpu.SemaphoreType.DMA,
        ],
    )
    def k(y3h, s3h, d3h, a3h0, a3h1, a3h2, a3h3, agg_h,
          src_v, dst_v, a_v, rows_v, msg_v, agg_sh,
          sl0, sl1, sg0, sg1, sda, sc0, sc1):
        c = lax.axis_index("c")
        s = lax.axis_index("s")
        wid = c * _NS + s
        sload = (sl0, sl1)
        sgat = (sg0, sg1)
        ssc = (sc0, sc1)

        def zero_body(i, carry):
            msg_v[0, i] = jnp.zeros((_D,), jnp.float32)
            msg_v[1, i] = jnp.zeros((_D,), jnp.float32)
            return carry

        lax.fori_loop(0, _CH, zero_body, 0)
        for r0 in range(0, rpt_i, _CH):
            w = min(_CH, rpt_i - r0)
            pltpu.sync_copy(msg_v.at[0, pl.ds(0, w)],
                            agg_sh.at[pl.ds(s * rpt_i + r0, w)])
        plsc.subcore_barrier()

        lanes = lax.iota(jnp.int32, _D)
        col0 = jnp.zeros((_D,), jnp.int32)

        def start_src(row, b):
            pltpu.async_copy(s3h.at[row], src_v.at[b], sload[b])

        def wait_src(b):
            pltpu.make_async_copy(s3h.at[0], src_v.at[b], sload[b]).wait()

        def start_gather(b):
            pltpu.async_copy(y3h.at[src_v.at[b]], rows_v.at[b], sgat[b])

        def wait_gather(b):
            pltpu.make_async_copy(y3h.at[pl.ds(0, _CH)], rows_v.at[b],
                                  sgat[b]).wait()

        ahs = (a3h0, a3h1, a3h2, a3h3)
        base = wid * _NCH
        start_src(base, 0)
        wait_src(0)
        start_gather(0)
        start_src(base + 1, 1)

        def wait_scatter(b):
            pltpu.make_async_copy(msg_v.at[b],
                                  agg_sh.at[dst_v.at[b]], ssc[b]).wait()

        def pair(j, carry):
            for b in (0, 1):
                cc = 2 * j + b
                row = base + cc

                @pl.when(cc >= 2)
                def _():
                    wait_scatter(b)

                pltpu.async_copy(d3h.at[row], dst_v.at[b], sda)
                for kk in range(_DE):
                    pltpu.async_copy(ahs[kk].at[row], a_v.at[kk], sda)

                @pl.when(cc + 1 < _NCH)
                def _():
                    wait_src(1 - b)
                    start_gather(1 - b)

                wait_gather(b)

                @pl.when(cc + 2 < _NCH)
                def _():
                    start_src(row + 2, b)

                pltpu.make_async_copy(d3h.at[0], dst_v.at[b], sda).wait()
                for kk in range(_DE):
                    pltpu.make_async_copy(ahs[kk].at[0], a_v.at[kk], sda).wait()

                def grp(g, carry2):
                    ev = g * _D + lanes
                    av0 = a_v[0, pl.ds(g * 16, 16)]
                    av1 = a_v[1, pl.ds(g * 16, 16)]
                    av2 = a_v[2, pl.ds(g * 16, 16)]
                    av3 = a_v[3, pl.ds(g * 16, 16)]
                    y0 = plsc.load_gather(rows_v, [col0 + b, ev, col0])
                    y1_ = plsc.load_gather(rows_v, [col0 + b, ev, col0 + 1])
                    y2_ = plsc.load_gather(rows_v, [col0 + b, ev, col0 + 2])
                    y3_ = plsc.load_gather(rows_v, [col0 + b, ev, col0 + 3])
                    y4_ = plsc.load_gather(rows_v, [col0 + b, ev, col0 + 4])
                    m = av0 * y0 + av1 * y1_ + av2 * y2_ + av3 * y3_ + y4_
                    plsc.store_scatter(msg_v, [col0 + b, ev, col0], m)
                    return carry2

                lax.fori_loop(0, _CH // _D, grp, 0)
                pltpu.async_copy(msg_v.at[b], agg_sh.at[dst_v.at[b]],
                                 ssc[b], add=True)
            return carry

        lax.fori_loop(0, _NCH // 2, pair, 0)
        wait_scatter(0)
        wait_scatter(1)

        plsc.subcore_barrier()
        pltpu.sync_copy(agg_sh.at[pl.ds(s * rpt_i, rpt_i)], agg_h.at[wid])

    return k(y3p, s3, d3, a3c[0], a3c[1], a3c[2], a3c[3])


def _tc_final(agg3_p, r3p, sel):
    npk = r3p.shape[0]

    def body(a3, r3r, selr, outo):
        m = jnp.dot(a3[0:npk, :] + a3[npk:2 * npk, :], selr[...],
                    preferred_element_type=jnp.float32)
        outo[...] = jax.nn.sigmoid(m + r3r[...])

    return pl.pallas_call(
        body,
        out_shape=jax.ShapeDtypeStruct((npk, 8), jnp.float32),
    )(agg3_p, r3p, sel)


_PB = 16384


def _tc_prep(ei, at, n_src, n_dst):
    e = ei.shape[1]
    npad = _NP - 10000

    def body(eir, ar, so, do_, ao0, ao1, ao2, ao3):
        i = pl.program_id(0)
        gcol = i * _PB + lax.broadcasted_iota(jnp.int32, (_PB,), 0)
        real = gcol < e
        padi = gcol - e
        so[...] = jnp.where(real, eir[0, :], n_src + padi % npad)
        do_[...] = jnp.where(real, eir[1, :], padi % n_dst)
        for k, o in enumerate((ao0, ao1, ao2, ao3)):
            o[...] = jnp.where(real, ar[k, :], 0.0)

    outs = pl.pallas_call(
        body,
        grid=(_EP // _PB,),
        in_specs=[pl.BlockSpec((2, _PB), lambda i: (0, i)),
                  pl.BlockSpec((_DE, _PB), lambda i: (0, i))],
        out_specs=[pl.BlockSpec((_PB,), lambda i: (i,))] * 6,
        out_shape=([jax.ShapeDtypeStruct((_EP,), jnp.int32)] * 2
                   + [jax.ShapeDtypeStruct((_EP,), jnp.float32)] * 4),
    )(ei, at)
    blk = [o.reshape(_EP // _CH, _CH) for o in outs]
    return blk[0], blk[1], tuple(blk[2:6])


def kernel(x_indivi, x_event, edge_index_e2i, edge_attr_e2i, edge_index_i2e,
           edge_attr_i2e, nnW1, nnb1, rootW1, b1, nnW2, nnb2, rootW2, b2,
           nnW3, nnb3, rootW3, b3):
    n_i, n_e = x_indivi.shape[0], x_event.shape[0]

    m1 = nnW1.reshape(_D, _D, _DE)
    c1 = jnp.concatenate([m1[:, :, k] for k in range(_DE)]
                         + [nnb1.reshape(_D, _D)], axis=1)
    m2 = nnW2.reshape(_D, _D, _DE)
    c2 = jnp.concatenate([m2[:, :, k] for k in range(_DE)]
                         + [nnb2.reshape(_D, _D)], axis=1)
    c3 = jnp.concatenate([nnW3, nnb3.reshape(_D, 1),
                          jnp.zeros((_D, _D - _DE - 1), jnp.float32)], axis=1)

    s1, d1, a1c = _tc_prep(edge_index_e2i.astype(jnp.int32),
                           edge_attr_e2i.T, n_e, n_i)
    s2, d2, a2c = _tc_prep(edge_index_i2e.astype(jnp.int32),
                           edge_attr_i2e.T, n_i, n_e)

    npk = n_i // 8
    eye8 = jnp.eye(8, dtype=jnp.float32)
    bd_r1 = jnp.kron(eye8, rootW1.T)
    bd_r2 = jnp.kron(eye8, rootW2.T)
    bd_c3 = jnp.kron(eye8, c3)
    bd_w3 = jnp.kron(eye8, rootW3.T)
    sel3 = jnp.kron(eye8, jnp.eye(_D, 1, dtype=jnp.float32))
    b1row = jnp.tile(b1, 8).reshape(1, 8 * _D)
    b2row = jnp.tile(b2, 8).reshape(1, 8 * _D)
    b3row = jnp.tile(b3, 8).reshape(1, 8)

    y1 = _tc_y(x_event, c1)
    y2 = _tc_y(x_indivi, c2)
    xip = x_indivi.reshape(npk, 8 * _D)
    xep = x_event.reshape(n_e // 8, 8 * _D)

    aggi = _sc_edge_pass(y1, s1, d1, a1c, n_i)
    agge = _sc_edge_pass(y2, s2, d2, a2c, n_e)
    aggi_p = aggi.reshape(_NC * npk, 8 * _D)
    agge_p = agge.reshape(_NC * (n_e // 8), 8 * _D)

    y3pk, r3p = _tc_mid(aggi_p, agge_p, xip, xep, bd_r1, bd_r2,
                        b1row, b2row, bd_c3, bd_w3, b3row)
    y3p = y3pk.reshape(_NP, _D)

    agg3_p = _sc_layer2(y3p, s1, d1, a1c, n_i).reshape(_NC * npk, 8 * _D)
    return _tc_final(agg3_p, r3p, sel3).reshape(n_i, 1)
